# baseline (speedup 1.0000x reference)
import functools

import jax
import jax.numpy as jnp
from jax.experimental import pallas as pl
from jax.experimental.pallas import tpu as pltpu

_BLK = 1000


def _fused_gru_kernel(x2_ref, wz_ref, bz_ref, wh_ref, bh_ref, wl_ref, bl_ref,
                      out_ref):
    x2 = x2_ref[...]
    f = x2.shape[1] // 2

    def head(x):
        z = jax.nn.sigmoid(
            jnp.dot(x, wz_ref[...], preferred_element_type=jnp.float32)
            + bz_ref[...])
        ht = jnp.tanh(
            jnp.dot(x, wh_ref[...], preferred_element_type=jnp.float32)
            + bh_ref[...])
        h = jax.nn.relu((1.0 - z) * ht)
        return (jnp.dot(h, wl_ref[...], preferred_element_type=jnp.float32)
                + bl_ref[...])

    o_even = head(x2[:, :f])
    o_odd = head(x2[:, f:])
    out_ref[...] = jnp.concatenate([o_even, o_odd], axis=1)


@functools.partial(jax.jit, static_argnames=())
def kernel(x, edge_index, edge_weight, W_xz, b_xz, W_hz, b_hz, W_xr, b_xr,
           W_hr, b_hr, W_xh, b_xh, W_hh, b_hh, W_lin, b_lin):
    n, f_in = x.shape
    out_len = W_lin.shape[1]
    half = n // 2
    bz = (b_xz + b_hz).reshape(1, -1)
    bh = (b_xh + b_hh).reshape(1, -1)
    bl = b_lin.reshape(1, -1)
    x2 = x.reshape(half, 2 * f_in)

    out2 = pl.pallas_call(
        _fused_gru_kernel,
        grid=(half // _BLK,),
        in_specs=[
            pl.BlockSpec((_BLK, 2 * f_in), lambda i: (i, 0)),
            pl.BlockSpec((f_in, W_xz.shape[1]), lambda i: (0, 0)),
            pl.BlockSpec((1, W_xz.shape[1]), lambda i: (0, 0)),
            pl.BlockSpec((f_in, W_xh.shape[1]), lambda i: (0, 0)),
            pl.BlockSpec((1, W_xh.shape[1]), lambda i: (0, 0)),
            pl.BlockSpec((W_lin.shape[0], out_len), lambda i: (0, 0)),
            pl.BlockSpec((1, out_len), lambda i: (0, 0)),
        ],
        out_specs=pl.BlockSpec((_BLK, 2 * out_len), lambda i: (i, 0)),
        out_shape=jax.ShapeDtypeStruct((half, 2 * out_len), x.dtype),
        compiler_params=pltpu.CompilerParams(
            dimension_semantics=("parallel",)),
    )(x2, W_xz, bz, W_xh, bh, W_lin, bl)
    return (out2.reshape(n, out_len),)


# speedup vs baseline: 1.5488x; 1.5488x over previous
import functools

import jax
import jax.numpy as jnp
from jax.experimental import pallas as pl
from jax.experimental.pallas import tpu as pltpu

_BLK = 2000


def _fused_gru_kernel(x_ref, wz_ref, bz_ref, wh_ref, bh_ref, wl_ref, bl_ref,
                      out_ref):
    x = x_ref[...]
    t = jnp.tanh(
        jnp.dot(x, wz_ref[...] * 0.5, preferred_element_type=jnp.float32)
        + bz_ref[...] * 0.5)
    ht = jnp.tanh(
        jnp.dot(x, wh_ref[...], preferred_element_type=jnp.float32)
        + bh_ref[...])
    h = (1.0 - t) * jax.nn.relu(ht)
    out_ref[...] = (
        jnp.dot(h, wl_ref[...] * 0.5, preferred_element_type=jnp.float32)
        + bl_ref[...])


@functools.partial(jax.jit, static_argnames=())
def kernel(x, edge_index, edge_weight, W_xz, b_xz, W_hz, b_hz, W_xr, b_xr,
           W_hr, b_hr, W_xh, b_xh, W_hh, b_hh, W_lin, b_lin):
    n, f_in = x.shape
    out_len = W_lin.shape[1]
    bz = (b_xz + b_hz).reshape(1, -1)
    bh = (b_xh + b_hh).reshape(1, -1)
    bl = b_lin.reshape(1, -1)

    out = pl.pallas_call(
        _fused_gru_kernel,
        grid=(n // _BLK,),
        in_specs=[
            pl.BlockSpec((_BLK, f_in), lambda i: (i, 0)),
            pl.BlockSpec((f_in, W_xz.shape[1]), lambda i: (0, 0)),
            pl.BlockSpec((1, W_xz.shape[1]), lambda i: (0, 0)),
            pl.BlockSpec((f_in, W_xh.shape[1]), lambda i: (0, 0)),
            pl.BlockSpec((1, W_xh.shape[1]), lambda i: (0, 0)),
            pl.BlockSpec((W_lin.shape[0], out_len), lambda i: (0, 0)),
            pl.BlockSpec((1, out_len), lambda i: (0, 0)),
        ],
        out_specs=pl.BlockSpec((_BLK, out_len), lambda i: (i, 0)),
        out_shape=jax.ShapeDtypeStruct((n, out_len), x.dtype),
        compiler_params=pltpu.CompilerParams(
            dimension_semantics=("parallel",)),
    )(x, W_xz, bz, W_xh, bh, W_lin, bl)
    return (out,)


# speedup vs baseline: 1.8875x; 1.2187x over previous
import functools

import jax
import jax.numpy as jnp
from jax.experimental import pallas as pl
from jax.experimental.pallas import tpu as pltpu

_BLK = 2048


def _fused_gru_kernel(x_ref, wz_ref, bz_ref, wh_ref, bh_ref, wl_ref, out_ref):
    x = x_ref[...]
    t = jnp.tanh(
        jnp.dot(x, wz_ref[...] * 0.5, preferred_element_type=jnp.float32)
        + bz_ref[...] * 0.5)
    ht = jnp.tanh(
        jnp.dot(x, wh_ref[...], preferred_element_type=jnp.float32)
        + bh_ref[...])
    h = (1.0 - t) * jax.nn.relu(ht)
    out_ref[...] = jax.lax.dot_general(
        wl_ref[...] * 0.5, h, (((0,), (1,)), ((), ())),
        preferred_element_type=jnp.float32)


@functools.partial(jax.jit, static_argnames=())
def kernel(x, edge_index, edge_weight, W_xz, b_xz, W_hz, b_hz, W_xr, b_xr,
           W_hr, b_hr, W_xh, b_xh, W_hh, b_hh, W_lin, b_lin):
    n, f_in = x.shape
    out_len = W_lin.shape[1]
    bz = (b_xz + b_hz).reshape(1, -1)
    bh = (b_xh + b_hh).reshape(1, -1)

    steps = pl.cdiv(n, _BLK)
    n_pad = steps * _BLK
    out_t = pl.pallas_call(
        _fused_gru_kernel,
        grid=(steps,),
        in_specs=[
            pl.BlockSpec((_BLK, f_in), lambda i: (i, 0)),
            pl.BlockSpec((f_in, W_xz.shape[1]), lambda i: (0, 0)),
            pl.BlockSpec((1, W_xz.shape[1]), lambda i: (0, 0)),
            pl.BlockSpec((f_in, W_xh.shape[1]), lambda i: (0, 0)),
            pl.BlockSpec((1, W_xh.shape[1]), lambda i: (0, 0)),
            pl.BlockSpec((W_lin.shape[0], out_len), lambda i: (0, 0)),
        ],
        out_specs=pl.BlockSpec((out_len, _BLK), lambda i: (0, i)),
        out_shape=jax.ShapeDtypeStruct((out_len, n_pad), x.dtype),
        compiler_params=pltpu.CompilerParams(
            dimension_semantics=("parallel",)),
    )(x, W_xz, bz, W_xh, bh, W_lin)
    return (out_t[:, :n].T + b_lin[None, :],)


# speedup vs baseline: 1.9933x; 1.0560x over previous
import functools

import jax
import jax.numpy as jnp
from jax.experimental import pallas as pl
from jax.experimental.pallas import tpu as pltpu

_BLK = 2560


def _fused_gru_kernel(x_ref, wz_ref, bz_ref, wh_ref, bh_ref, wl_ref, out_ref):
    x = x_ref[...]
    t = jnp.tanh(
        jnp.dot(x, wz_ref[...] * 0.5, preferred_element_type=jnp.float32)
        + bz_ref[...] * 0.5)
    ht = jnp.tanh(
        jnp.dot(x, wh_ref[...], preferred_element_type=jnp.float32)
        + bh_ref[...])
    h = (1.0 - t) * jax.nn.relu(ht)
    out_ref[...] = jax.lax.dot_general(
        wl_ref[...] * 0.5, h, (((0,), (1,)), ((), ())),
        preferred_element_type=jnp.float32)


@functools.partial(jax.jit, static_argnames=())
def kernel(x, edge_index, edge_weight, W_xz, b_xz, W_hz, b_hz, W_xr, b_xr,
           W_hr, b_hr, W_xh, b_xh, W_hh, b_hh, W_lin, b_lin):
    n, f_in = x.shape
    out_len = W_lin.shape[1]
    bz = (b_xz + b_hz).reshape(1, -1)
    bh = (b_xh + b_hh).reshape(1, -1)

    steps = pl.cdiv(n, _BLK)
    n_pad = steps * _BLK
    out_t = pl.pallas_call(
        _fused_gru_kernel,
        grid=(steps,),
        in_specs=[
            pl.BlockSpec((_BLK, f_in), lambda i: (i, 0)),
            pl.BlockSpec((f_in, W_xz.shape[1]), lambda i: (0, 0)),
            pl.BlockSpec((1, W_xz.shape[1]), lambda i: (0, 0)),
            pl.BlockSpec((f_in, W_xh.shape[1]), lambda i: (0, 0)),
            pl.BlockSpec((1, W_xh.shape[1]), lambda i: (0, 0)),
            pl.BlockSpec((W_lin.shape[0], out_len), lambda i: (0, 0)),
        ],
        out_specs=pl.BlockSpec((out_len, _BLK), lambda i: (0, i)),
        out_shape=jax.ShapeDtypeStruct((out_len, n_pad), x.dtype),
        compiler_params=pltpu.CompilerParams(
            dimension_semantics=("parallel",)),
    )(x, W_xz, bz, W_xh, bh, W_lin)
    return (out_t[:, :n].T + b_lin[None, :],)


# speedup vs baseline: 2.1217x; 1.0644x over previous
import functools

import jax
import jax.numpy as jnp
from jax.experimental import pallas as pl
from jax.experimental.pallas import tpu as pltpu

_BLK = 5120


def _fused_gru_kernel(x_ref, wz_ref, bz_ref, wh_ref, bh_ref, wl_ref, out_ref):
    x = x_ref[...]
    t = jnp.tanh(
        jnp.dot(x, wz_ref[...] * 0.5, preferred_element_type=jnp.float32)
        + bz_ref[...] * 0.5)
    ht = jnp.tanh(
        jnp.dot(x, wh_ref[...], preferred_element_type=jnp.float32)
        + bh_ref[...])
    h = (1.0 - t) * jax.nn.relu(ht)
    out_ref[...] = jax.lax.dot_general(
        wl_ref[...] * 0.5, h, (((0,), (1,)), ((), ())),
        preferred_element_type=jnp.float32)


@functools.partial(jax.jit, static_argnames=())
def kernel(x, edge_index, edge_weight, W_xz, b_xz, W_hz, b_hz, W_xr, b_xr,
           W_hr, b_hr, W_xh, b_xh, W_hh, b_hh, W_lin, b_lin):
    n, f_in = x.shape
    out_len = W_lin.shape[1]
    bz = (b_xz + b_hz).reshape(1, -1)
    bh = (b_xh + b_hh).reshape(1, -1)

    steps = pl.cdiv(n, _BLK)
    n_pad = steps * _BLK
    out_t = pl.pallas_call(
        _fused_gru_kernel,
        grid=(steps,),
        in_specs=[
            pl.BlockSpec((_BLK, f_in), lambda i: (i, 0)),
            pl.BlockSpec((f_in, W_xz.shape[1]), lambda i: (0, 0)),
            pl.BlockSpec((1, W_xz.shape[1]), lambda i: (0, 0)),
            pl.BlockSpec((f_in, W_xh.shape[1]), lambda i: (0, 0)),
            pl.BlockSpec((1, W_xh.shape[1]), lambda i: (0, 0)),
            pl.BlockSpec((W_lin.shape[0], out_len), lambda i: (0, 0)),
        ],
        out_specs=pl.BlockSpec((out_len, _BLK), lambda i: (0, i)),
        out_shape=jax.ShapeDtypeStruct((out_len, n_pad), x.dtype),
        compiler_params=pltpu.CompilerParams(
            dimension_semantics=("parallel",)),
    )(x, W_xz, bz, W_xh, bh, W_lin)
    return (out_t[:, :n].T + b_lin[None, :],)
